# baseline (speedup 1.0000x reference)
import functools

import jax
import jax.numpy as jnp
from jax import lax
from jax.experimental import pallas as pl
from jax.experimental.pallas import tpu as pltpu
from jax.experimental.pallas import tpu_sc as plsc

_LANES = 16
_CHUNK = 256


def _mlp_body(x_ref, w1_ref, b1_ref, w2_ref, b2_ref, logits_ref):
    h = jnp.tanh(
        jnp.dot(x_ref[...], w1_ref[...], preferred_element_type=jnp.float32)
        + b1_ref[...]
    )
    logits_ref[...] = (
        jnp.dot(h, w2_ref[...], preferred_element_type=jnp.float32) + b2_ref[...]
    )


def _tc_logits(x, W1, b1, W2, b2):
    n, d = x.shape
    h_dim = W1.shape[1]
    e = W2.shape[1]
    bm = 4096
    return pl.pallas_call(
        _mlp_body,
        grid=(n // bm,),
        in_specs=[
            pl.BlockSpec((bm, d), lambda i: (i, 0)),
            pl.BlockSpec((d, h_dim), lambda i: (0, 0)),
            pl.BlockSpec((1, h_dim), lambda i: (0, 0)),
            pl.BlockSpec((h_dim, e), lambda i: (0, 0)),
            pl.BlockSpec((1, e), lambda i: (0, 0)),
        ],
        out_specs=pl.BlockSpec((bm, e), lambda i: (i, 0)),
        out_shape=jax.ShapeDtypeStruct((n, e), jnp.float32),
    )(x, W1, b1.reshape(1, -1), W2, b2.reshape(1, -1))


def _sc_gates_kernel(n_tokens, num_experts):
    info = plsc.get_sparse_core_info()
    nc, ns = info.num_cores, info.num_subcores
    n_workers = nc * ns
    per_worker = n_tokens // n_workers
    n_chunks = per_worker // _CHUNK
    groups_per_chunk = _CHUNK // _LANES
    mesh = plsc.VectorSubcoreMesh(core_axis_name="c", subcore_axis_name="s")

    @functools.partial(
        pl.kernel,
        out_type=jax.ShapeDtypeStruct((n_tokens, num_experts), jnp.float32),
        mesh=mesh,
        scratch_types=[
            pltpu.VMEM((_CHUNK, num_experts), jnp.float32),
            pltpu.VMEM((_CHUNK, num_experts), jnp.float32),
            pltpu.VMEM((_CHUNK, num_experts), jnp.float32),
            pltpu.SemaphoreType.DMA,
            pltpu.SemaphoreType.DMA,
        ],
        compiler_params=pltpu.CompilerParams(needs_layout_passes=False),
    )
    def gates_kernel(logits_hbm, gates_hbm, lbuf0, lbuf1, gbuf, si0, si1):
        wid = lax.axis_index("s") * nc + lax.axis_index("c")
        lane = lax.iota(jnp.int32, _LANES)
        lb, si = [lbuf0, lbuf1], [si0, si1]
        base0 = wid * per_worker

        def compute_chunk(lbuf, gbuf):
            @plsc.parallel_loop(0, groups_per_chunk, unroll=4)
            def group_body(g):
                rows = g * _LANES + lane
                neg_inf = jnp.full((_LANES,), -jnp.inf, jnp.float32)
                m1 = [neg_inf] * 4
                m2 = [neg_inf] * 4
                for e in range(num_experts):
                    a = e % 4
                    ecol = jnp.bitwise_and(lane + e, num_experts - 1)
                    v = plsc.load_gather(lbuf, [rows, ecol])
                    m2[a] = jnp.maximum(m2[a], jnp.minimum(v, m1[a]))
                    m1[a] = jnp.maximum(m1[a], v)

                def merge(p, q):
                    hi = jnp.maximum(p[0], q[0])
                    lo = jnp.maximum(jnp.minimum(p[0], q[0]), jnp.maximum(p[1], q[1]))
                    return hi, lo

                top = merge(merge((m1[0], m2[0]), (m1[1], m2[1])),
                            merge((m1[2], m2[2]), (m1[3], m2[3])))
                big1, big2 = top
                e2 = jnp.exp(big2 - big1)
                g1 = 1.0 / (1.0 + e2)
                g2 = 1.0 - g1
                zero = jnp.zeros((_LANES,), jnp.float32)
                for e in range(num_experts):
                    ecol = jnp.bitwise_and(lane + e, num_experts - 1)
                    v = plsc.load_gather(lbuf, [rows, ecol])
                    col = jnp.where(v == big1, g1, jnp.where(v == big2, g2, zero))
                    plsc.store_scatter(gbuf, [rows, ecol], col)

        pltpu.async_copy(logits_hbm.at[pl.ds(base0, _CHUNK)], lb[0], si[0])

        def pair_body(i, _):
            for b in (0, 1):
                c = i * 2 + b
                src = logits_hbm.at[pl.ds(base0 + c * _CHUNK, _CHUNK)]
                pltpu.make_async_copy(src, lb[b], si[b]).wait()
                nxt = c + 1

                @pl.when(nxt < n_chunks)
                def _():
                    pltpu.async_copy(
                        logits_hbm.at[pl.ds(base0 + nxt * _CHUNK, _CHUNK)],
                        lb[1 - b],
                        si[1 - b],
                    )

                compute_chunk(lb[b], gbuf)
                pltpu.sync_copy(
                    gbuf, gates_hbm.at[pl.ds(base0 + c * _CHUNK, _CHUNK)]
                )
            return 0

        lax.fori_loop(0, n_chunks // 2, pair_body, 0)

    return gates_kernel


def kernel(x, W1, b1, W2, b2):
    n = x.shape[0]
    e = W2.shape[1]
    logits = _tc_logits(x, W1, b1, W2, b2)
    gates = _sc_gates_kernel(n, e)(logits)
    return (gates, logits)


# speedup vs baseline: 1.2226x; 1.2226x over previous
import functools

import jax
import jax.numpy as jnp
from jax import lax
from jax.experimental import pallas as pl
from jax.experimental.pallas import tpu as pltpu
from jax.experimental.pallas import tpu_sc as plsc

_LANES = 16
_CHUNK = 256


def _mlp_body(x_ref, w1_ref, b1_ref, w2_ref, b2_ref, logits_ref):
    h = jnp.tanh(
        jnp.dot(x_ref[...], w1_ref[...], preferred_element_type=jnp.float32)
        + b1_ref[...]
    )
    logits_ref[...] = (
        jnp.dot(h, w2_ref[...], preferred_element_type=jnp.float32) + b2_ref[...]
    )


def _tc_logits(x, W1, b1, W2, b2):
    n, d = x.shape
    h_dim = W1.shape[1]
    e = W2.shape[1]
    bm = 4096
    return pl.pallas_call(
        _mlp_body,
        grid=(n // bm,),
        in_specs=[
            pl.BlockSpec((bm, d), lambda i: (i, 0)),
            pl.BlockSpec((d, h_dim), lambda i: (0, 0)),
            pl.BlockSpec((1, h_dim), lambda i: (0, 0)),
            pl.BlockSpec((h_dim, e), lambda i: (0, 0)),
            pl.BlockSpec((1, e), lambda i: (0, 0)),
        ],
        out_specs=pl.BlockSpec((bm, e), lambda i: (i, 0)),
        out_shape=jax.ShapeDtypeStruct((n, e), jnp.float32),
    )(x, W1, b1.reshape(1, -1), W2, b2.reshape(1, -1))


def _sc_gates_kernel(n_tokens, num_experts):
    info = plsc.get_sparse_core_info()
    nc, ns = info.num_cores, info.num_subcores
    n_workers = nc * ns
    per_worker = n_tokens // n_workers
    n_chunks = per_worker // _CHUNK
    groups_per_chunk = _CHUNK // _LANES
    mesh = plsc.VectorSubcoreMesh(core_axis_name="c", subcore_axis_name="s")

    @functools.partial(
        pl.kernel,
        out_type=jax.ShapeDtypeStruct((n_tokens, num_experts), jnp.float32),
        mesh=mesh,
        scratch_types=[
            pltpu.VMEM((_CHUNK, num_experts), jnp.float32),
            pltpu.VMEM((_CHUNK, num_experts), jnp.float32),
            pltpu.VMEM((_CHUNK, num_experts), jnp.float32),
            pltpu.SemaphoreType.DMA,
            pltpu.SemaphoreType.DMA,
        ],
        compiler_params=pltpu.CompilerParams(needs_layout_passes=False),
    )
    def gates_kernel(logits_hbm, gates_hbm, lbuf0, lbuf1, gbuf, si0, si1):
        wid = lax.axis_index("s") * nc + lax.axis_index("c")
        lane = lax.iota(jnp.int32, _LANES)
        lb, si = [lbuf0, lbuf1], [si0, si1]
        base0 = wid * per_worker

        def compute_chunk(lbuf, gbuf):
            @plsc.parallel_loop(0, groups_per_chunk, unroll=2)
            def group_body(g):
                rows = g * _LANES + lane
                neg_inf = jnp.full((_LANES,), -jnp.inf, jnp.float32)
                m1 = [neg_inf] * 4
                m2 = [neg_inf] * 4
                for e in range(num_experts):
                    a = e % 4
                    ecol = jnp.bitwise_and(lane + e, num_experts - 1)
                    v = plsc.load_gather(lbuf, [rows, ecol])
                    m2[a] = jnp.maximum(m2[a], jnp.minimum(v, m1[a]))
                    m1[a] = jnp.maximum(m1[a], v)

                def merge(p, q):
                    hi = jnp.maximum(p[0], q[0])
                    lo = jnp.maximum(jnp.minimum(p[0], q[0]), jnp.maximum(p[1], q[1]))
                    return hi, lo

                top = merge(merge((m1[0], m2[0]), (m1[1], m2[1])),
                            merge((m1[2], m2[2]), (m1[3], m2[3])))
                big1, big2 = top
                e2 = jnp.exp(big2 - big1)
                g1 = 1.0 / (1.0 + e2)
                g2 = 1.0 - g1
                zero = jnp.zeros((_LANES,), jnp.float32)
                for e in range(num_experts):
                    ecol = jnp.bitwise_and(lane + e, num_experts - 1)
                    v = plsc.load_gather(lbuf, [rows, ecol])
                    col = jnp.where(v == big1, g1, jnp.where(v == big2, g2, zero))
                    plsc.store_scatter(gbuf, [rows, ecol], col)

        pltpu.async_copy(logits_hbm.at[pl.ds(base0, _CHUNK)], lb[0], si[0])

        def pair_body(i, _):
            for b in (0, 1):
                c = i * 2 + b
                src = logits_hbm.at[pl.ds(base0 + c * _CHUNK, _CHUNK)]
                pltpu.make_async_copy(src, lb[b], si[b]).wait()
                nxt = c + 1

                @pl.when(nxt < n_chunks)
                def _():
                    pltpu.async_copy(
                        logits_hbm.at[pl.ds(base0 + nxt * _CHUNK, _CHUNK)],
                        lb[1 - b],
                        si[1 - b],
                    )

                compute_chunk(lb[b], gbuf)
                pltpu.sync_copy(
                    gbuf, gates_hbm.at[pl.ds(base0 + c * _CHUNK, _CHUNK)]
                )
            return 0

        lax.fori_loop(0, n_chunks // 2, pair_body, 0)

    return gates_kernel


def kernel(x, W1, b1, W2, b2):
    n = x.shape[0]
    e = W2.shape[1]
    logits = _tc_logits(x, W1, b1, W2, b2)
    gates = _sc_gates_kernel(n, e)(logits)
    return (gates, logits)
